# baseline (speedup 1.0000x reference)
import jax
import jax.numpy as jnp
from jax.experimental import pallas as pl
from jax.experimental.pallas import tpu as pltpu

_BLOCK = 5000
_NEG = -jnp.inf


def _mm(a, w):
    return jax.lax.dot_general(a, w, (((1,), (0,)), ((), ())),
                               preferred_element_type=jnp.float32)


def _mm16(a, w):
    return jax.lax.dot_general(a, w, (((1,), (0,)), ((), ())),
                               precision=jax.lax.Precision.DEFAULT,
                               preferred_element_type=jnp.float32)


def _bulk_kernel(x_ref, W1p, b1p, W2p, b2p, sW3, cW3,
                 smax_out, cmax_out,
                 smax, cmax):
    i = pl.program_id(1)
    nsteps = pl.num_programs(1)
    xb = x_ref[...]

    def gmax(h):
        return jnp.max(h.reshape(_BLOCK // 8, 8, 1024), axis=0)

    h = jnp.maximum(_mm16(xb, W1p[...]) + b1p[...], 0).astype(jnp.bfloat16)
    h = jnp.maximum(_mm16(h, W2p[...]) + b2p[...], 0).astype(jnp.bfloat16)
    bs = gmax(_mm16(h[:, 0:128], sW3[...]))
    bc = gmax(_mm16(h[:, 128:256], cW3[...]))

    @pl.when(i == 0)
    def _init():
        smax[...] = jnp.full((8, 1024), _NEG, jnp.float32)
        cmax[...] = jnp.full((8, 1024), _NEG, jnp.float32)

    smax[...] = jnp.maximum(smax[...], bs)
    cmax[...] = jnp.maximum(cmax[...], bc)

    @pl.when(i == nsteps - 1)
    def _out():
        smax_out[...] = smax[...]
        cmax_out[...] = cmax[...]


def _tail_kernel(x8_ref,
                 sWr1, sWn1, sb1, sWr2, sWn2, sb2, sWr3, sWn3, sb3,
                 fc1W, fc1b, fc2W, fc2b, fc3W, fc3b,
                 cWr1, cWn1, cb1, cWr2, cWn2, cb2, cWr3, cWn3, cb3,
                 smax_ref, cmax_ref,
                 h_out, t9_out):
    x8 = x8_ref[...]
    rows = jax.lax.broadcasted_iota(jnp.int32, (8, 1), 0)
    sel = (rows == 1).astype(jnp.float32)
    keep = rows < 2

    def gconv(h, wr, wn):
        return _mm(h, wr[...]) + sel * _mm(h[0:1, :], wn[...])

    e = jnp.maximum(gconv(x8, sWr1, sWn1) + sb1[...], 0.0)
    e = jnp.maximum(gconv(e, sWr2, sWn2) + sb2[...], 0.0)
    e = gconv(e, sWr3, sWn3)
    es = jnp.max(jnp.where(keep, e, _NEG), axis=0, keepdims=True)
    e = jnp.maximum(gconv(x8, cWr1, cWn1) + cb1[...], 0.0)
    e = jnp.maximum(gconv(e, cWr2, cWn2) + cb2[...], 0.0)
    e = gconv(e, cWr3, cWn3)
    ec = jnp.max(jnp.where(keep, e, _NEG), axis=0, keepdims=True)

    bulk_s = jnp.max(smax_ref[...].astype(jnp.float32), axis=0, keepdims=True)
    bulk_c = jnp.max(cmax_ref[...].astype(jnp.float32), axis=0, keepdims=True)
    h_out[...] = jnp.maximum(bulk_c, ec) + cb3[...]
    s = jnp.maximum(jnp.maximum(bulk_s, es) + sb3[...], 0.0)
    t = jnp.maximum(_mm(s, fc1W[...]) + fc1b[...], 0.0)
    t = jnp.maximum(_mm(t, fc2W[...]) + fc2b[...], 0.0)
    t9 = _mm(t, fc3W[...]) + fc3b[...]
    col = jax.lax.broadcasted_iota(jnp.int32, (1, 9), 1)
    t9_out[...] = t9 + (col % 4 == 0).astype(jnp.float32)


def kernel(x, stn_g1_Wr, stn_g1_Wn, stn_g1_b, stn_g2_Wr, stn_g2_Wn, stn_g2_b,
           stn_g3_Wr, stn_g3_Wn, stn_g3_b, stn_fc1_W, stn_fc1_b,
           stn_fc2_W, stn_fc2_b, stn_fc3_W, stn_fc3_b,
           c1_Wr, c1_Wn, c1_b, c2_Wr, c2_Wn, c2_b, c3_Wr, c3_Wn, c3_b):
    n = x.shape[0]
    grid = n // _BLOCK
    assert grid * _BLOCK == n
    bf = jnp.bfloat16

    x8 = x[0:8]
    x16 = x.at[1].set(x[0]).astype(bf)

    W1p = jnp.concatenate([stn_g1_Wr, c1_Wr], axis=1).astype(bf)
    b1p = jnp.concatenate([stn_g1_b, c1_b]).reshape(1, 128).astype(bf)
    z = jnp.zeros((64, 128), jnp.float32)
    W2p = jnp.concatenate([
        jnp.concatenate([stn_g2_Wr, z], axis=1),
        jnp.concatenate([z, c2_Wr], axis=1),
    ], axis=0).astype(bf)
    b2p = jnp.concatenate([stn_g2_b, c2_b]).reshape(1, 256).astype(bf)
    sW3 = stn_g3_Wr.astype(bf)
    cW3 = c3_Wr.astype(bf)

    bulk_w = (W1p, b1p, W2p, b2p, sW3, cW3)
    bspecs = [pl.BlockSpec(w.shape, lambda i, j: (0, 0)) for w in bulk_w]

    grid2 = grid // 2
    assert grid2 * 2 == grid
    smax, cmax = pl.pallas_call(
        _bulk_kernel,
        grid=(2, grid2),
        in_specs=[pl.BlockSpec((_BLOCK, 3),
                               lambda i, j: (i * grid2 + j, 0))] + bspecs,
        out_specs=[pl.BlockSpec((8, 1024), lambda i, j: (i, 0)),
                   pl.BlockSpec((8, 1024), lambda i, j: (i, 0))],
        out_shape=[jax.ShapeDtypeStruct((16, 1024), jnp.float32),
                   jax.ShapeDtypeStruct((16, 1024), jnp.float32)],
        scratch_shapes=[pltpu.VMEM((8, 1024), jnp.float32),
                        pltpu.VMEM((8, 1024), jnp.float32)],
        compiler_params=pltpu.CompilerParams(
            dimension_semantics=("parallel", "arbitrary")),
    )(x16, *bulk_w)

    row = lambda v: v.reshape(1, -1)
    tail_in = (
        x8,
        stn_g1_Wr, stn_g1_Wn, row(stn_g1_b),
        stn_g2_Wr, stn_g2_Wn, row(stn_g2_b),
        stn_g3_Wr, stn_g3_Wn, row(stn_g3_b),
        stn_fc1_W, row(stn_fc1_b), stn_fc2_W, row(stn_fc2_b),
        stn_fc3_W, row(stn_fc3_b),
        c1_Wr, c1_Wn, row(c1_b),
        c2_Wr, c2_Wn, row(c2_b),
        c3_Wr, c3_Wn, row(c3_b),
        smax, cmax,
    )
    h, t9 = pl.pallas_call(
        _tail_kernel,
        out_shape=[jax.ShapeDtypeStruct((1, 1024), jnp.float32),
                   jax.ShapeDtypeStruct((1, 9), jnp.float32)],
    )(*tail_in)
    return h, t9.reshape(3, 3)


# speedup vs baseline: 1.0141x; 1.0141x over previous
import jax
import jax.numpy as jnp
from jax.experimental import pallas as pl
from jax.experimental.pallas import tpu as pltpu

_BLOCK = 4000
_NEG = -jnp.inf


def _mm(a, w):
    return jax.lax.dot_general(a, w, (((1,), (0,)), ((), ())),
                               preferred_element_type=jnp.float32)


def _mm16(a, w):
    return jax.lax.dot_general(a, w, (((1,), (0,)), ((), ())),
                               precision=jax.lax.Precision.DEFAULT,
                               preferred_element_type=jnp.float32)


def _bulk_kernel(x_ref, W1p, b1p, W2p, b2p, sW3, cW3,
                 smax_out, cmax_out,
                 smax, cmax):
    i = pl.program_id(0)
    nsteps = pl.num_programs(0)
    xb = x_ref[...]

    def gmax(h):
        return jnp.max(h.reshape(_BLOCK // 8, 8, 1024), axis=0)

    h = jnp.maximum(_mm16(xb, W1p[...]) + b1p[...], 0).astype(jnp.bfloat16)
    h = jnp.maximum(_mm16(h, W2p[...]) + b2p[...], 0).astype(jnp.bfloat16)
    bs = gmax(_mm16(h[:, 0:128], sW3[...]))
    bc = gmax(_mm16(h[:, 128:256], cW3[...]))

    @pl.when(i == 0)
    def _init():
        smax[...] = jnp.full((8, 1024), _NEG, jnp.float32)
        cmax[...] = jnp.full((8, 1024), _NEG, jnp.float32)

    smax[...] = jnp.maximum(smax[...], bs)
    cmax[...] = jnp.maximum(cmax[...], bc)

    @pl.when(i == nsteps - 1)
    def _out():
        smax_out[...] = smax[...]
        cmax_out[...] = cmax[...]


def _tail_kernel(x8_ref,
                 sWr1, sWn1, sb1, sWr2, sWn2, sb2, sWr3, sWn3, sb3,
                 fc1W, fc1b, fc2W, fc2b, fc3W, fc3b,
                 cWr1, cWn1, cb1, cWr2, cWn2, cb2, cWr3, cWn3, cb3,
                 smax_ref, cmax_ref,
                 h_out, t9_out):
    x8 = x8_ref[...]
    rows = jax.lax.broadcasted_iota(jnp.int32, (8, 1), 0)
    sel = (rows == 1).astype(jnp.float32)
    keep = rows < 2

    def gconv(h, wr, wn):
        return _mm(h, wr[...]) + sel * _mm(h[0:1, :], wn[...])

    e = jnp.maximum(gconv(x8, sWr1, sWn1) + sb1[...], 0.0)
    e = jnp.maximum(gconv(e, sWr2, sWn2) + sb2[...], 0.0)
    e = gconv(e, sWr3, sWn3)
    es = jnp.max(jnp.where(keep, e, _NEG), axis=0, keepdims=True)
    e = jnp.maximum(gconv(x8, cWr1, cWn1) + cb1[...], 0.0)
    e = jnp.maximum(gconv(e, cWr2, cWn2) + cb2[...], 0.0)
    e = gconv(e, cWr3, cWn3)
    ec = jnp.max(jnp.where(keep, e, _NEG), axis=0, keepdims=True)

    bulk_s = jnp.max(smax_ref[...].astype(jnp.float32), axis=0, keepdims=True)
    bulk_c = jnp.max(cmax_ref[...].astype(jnp.float32), axis=0, keepdims=True)
    h_out[...] = jnp.maximum(bulk_c, ec) + cb3[...]
    s = jnp.maximum(jnp.maximum(bulk_s, es) + sb3[...], 0.0)
    t = jnp.maximum(_mm(s, fc1W[...]) + fc1b[...], 0.0)
    t = jnp.maximum(_mm(t, fc2W[...]) + fc2b[...], 0.0)
    t9 = _mm(t, fc3W[...]) + fc3b[...]
    col = jax.lax.broadcasted_iota(jnp.int32, (1, 9), 1)
    t9_out[...] = t9 + (col % 4 == 0).astype(jnp.float32)


def kernel(x, stn_g1_Wr, stn_g1_Wn, stn_g1_b, stn_g2_Wr, stn_g2_Wn, stn_g2_b,
           stn_g3_Wr, stn_g3_Wn, stn_g3_b, stn_fc1_W, stn_fc1_b,
           stn_fc2_W, stn_fc2_b, stn_fc3_W, stn_fc3_b,
           c1_Wr, c1_Wn, c1_b, c2_Wr, c2_Wn, c2_b, c3_Wr, c3_Wn, c3_b):
    n = x.shape[0]
    grid = n // _BLOCK
    assert grid * _BLOCK == n
    bf = jnp.bfloat16

    x8 = x[0:8]
    x16 = x.at[1].set(x[0]).astype(bf)

    W1p = jnp.concatenate([stn_g1_Wr, c1_Wr], axis=1).astype(bf)
    b1p = jnp.concatenate([stn_g1_b, c1_b]).reshape(1, 128).astype(bf)
    z = jnp.zeros((64, 128), jnp.float32)
    W2p = jnp.concatenate([
        jnp.concatenate([stn_g2_Wr, z], axis=1),
        jnp.concatenate([z, c2_Wr], axis=1),
    ], axis=0).astype(bf)
    b2p = jnp.concatenate([stn_g2_b, c2_b]).reshape(1, 256).astype(bf)
    sW3 = stn_g3_Wr.astype(bf)
    cW3 = c3_Wr.astype(bf)

    bulk_w = (W1p, b1p, W2p, b2p, sW3, cW3)
    bspecs = [pl.BlockSpec(w.shape, lambda i: (0, 0)) for w in bulk_w]

    smax, cmax = pl.pallas_call(
        _bulk_kernel,
        grid=(grid,),
        in_specs=[pl.BlockSpec((_BLOCK, 3), lambda i: (i, 0))] + bspecs,
        out_specs=[pl.BlockSpec((8, 1024), lambda i: (0, 0)),
                   pl.BlockSpec((8, 1024), lambda i: (0, 0))],
        out_shape=[jax.ShapeDtypeStruct((8, 1024), jnp.float32),
                   jax.ShapeDtypeStruct((8, 1024), jnp.float32)],
        scratch_shapes=[pltpu.VMEM((8, 1024), jnp.float32),
                        pltpu.VMEM((8, 1024), jnp.float32)],
    )(x16, *bulk_w)

    row = lambda v: v.reshape(1, -1)
    tail_in = (
        x8,
        stn_g1_Wr, stn_g1_Wn, row(stn_g1_b),
        stn_g2_Wr, stn_g2_Wn, row(stn_g2_b),
        stn_g3_Wr, stn_g3_Wn, row(stn_g3_b),
        stn_fc1_W, row(stn_fc1_b), stn_fc2_W, row(stn_fc2_b),
        stn_fc3_W, row(stn_fc3_b),
        c1_Wr, c1_Wn, row(c1_b),
        c2_Wr, c2_Wn, row(c2_b),
        c3_Wr, c3_Wn, row(c3_b),
        smax, cmax,
    )
    h, t9 = pl.pallas_call(
        _tail_kernel,
        out_shape=[jax.ShapeDtypeStruct((1, 1024), jnp.float32),
                   jax.ShapeDtypeStruct((1, 9), jnp.float32)],
    )(*tail_in)
    return h, t9.reshape(3, 3)
